# baseline (speedup 1.0000x reference)
import functools
import jax
import jax.numpy as jnp
from jax import lax
from jax.experimental import pallas as pl
from jax.experimental.pallas import tpu as pltpu
from jax.experimental.pallas import tpu_sc as plsc

N = 16384
C = 4096
K = int(0.7 * N)

NSC = 4096
NTC = N - NSC
NW = 32
RPW = NSC // NW
CH = 8
NCH = RPW // CH
RB = 1024
NBLK = NTC // RB
TLROWS = NTC // 128
SLROWS = NSC // 128

_mesh = plsc.VectorSubcoreMesh(core_axis_name="c", subcore_axis_name="s")


@functools.partial(
    pl.kernel,
    mesh=_mesh,
    out_type=(
        jax.ShapeDtypeStruct((NW, RPW * 16), jnp.float32),
        jax.ShapeDtypeStruct((NW, RPW), jnp.float32),
    ),
    scratch_types=[
        pltpu.VMEM((CH * C,), jnp.float32),
        pltpu.VMEM((CH * C,), jnp.float32),
        pltpu.VMEM((RPW,), jnp.int32),
        pltpu.VMEM((RPW,), jnp.int32),
        pltpu.VMEM((RPW * 16,), jnp.float32),
        pltpu.VMEM((RPW,), jnp.float32),
        pltpu.SemaphoreType.DMA,
        pltpu.SemaphoreType.DMA,
        pltpu.SemaphoreType.DMA,
    ],
)
def _sc_rows(pred_flat, tgt_hbm, s16_out, xt_out, buf0, buf1, tgt_v, idx_buf,
             s16buf, xt_v, sem0, sem1, gsem):
    cid = lax.axis_index("c")
    sid = lax.axis_index("s")
    wid = sid * 2 + cid
    base = NTC + wid * RPW
    ebase = base * C

    pltpu.sync_copy(tgt_hbm.at[pl.ds(base, RPW)], tgt_v)
    lanes = lax.iota(jnp.int32, 16)

    def build_idx(g, carry):
        tvec = tgt_v[pl.ds(g * 16, 16)]
        idx_buf[pl.ds(g * 16, 16)] = (base + g * 16 + lanes) * C + tvec
        return carry

    lax.fori_loop(0, RPW // 16, build_idx, 0)
    pltpu.async_copy(pred_flat.at[idx_buf], xt_v, gsem)

    def process(buf, ch_idx):
        for r in range(CH):
            z = jnp.zeros((16,), jnp.float32)

            def col_body(j, accs):
                a0, a1, a2, a3 = accs
                b = r * C + j * 64
                a0 = a0 + jnp.exp(buf[pl.ds(b, 16)])
                a1 = a1 + jnp.exp(buf[pl.ds(b + 16, 16)])
                a2 = a2 + jnp.exp(buf[pl.ds(b + 32, 16)])
                a3 = a3 + jnp.exp(buf[pl.ds(b + 48, 16)])
                return (a0, a1, a2, a3)

            a0, a1, a2, a3 = lax.fori_loop(0, C // 64, col_body, (z, z, z, z))
            s16buf[pl.ds((ch_idx * CH + r) * 16, 16)] = (a0 + a1) + (a2 + a3)

    CB = CH * C
    pltpu.async_copy(pred_flat.at[pl.ds(ebase, CB)], buf0, sem0)

    def outer(j, carry):
        c0 = 2 * j
        e0 = ebase + c0 * CB
        pltpu.async_copy(pred_flat.at[pl.ds(e0 + CB, CB)], buf1, sem1)
        pltpu.make_async_copy(pred_flat.at[pl.ds(e0, CB)], buf0, sem0).wait()
        process(buf0, c0)

        @pl.when(c0 + 2 < NCH)
        def _():
            pltpu.async_copy(pred_flat.at[pl.ds(e0 + 2 * CB, CB)], buf0, sem0)

        pltpu.make_async_copy(pred_flat.at[pl.ds(e0 + CB, CB)], buf1, sem1).wait()
        process(buf1, c0 + 1)
        return carry

    lax.fori_loop(0, NCH // 2, outer, 0)
    pltpu.make_async_copy(pred_flat.at[idx_buf], xt_v, gsem).wait()
    pltpu.sync_copy(s16buf, s16_out.at[wid])
    pltpu.sync_copy(xt_v, xt_out.at[wid])


def _tc_body(pred_ref, tgt_ref, loss_ref):
    i = pl.program_id(0)
    x = pred_ref[...]
    s = jnp.sum(jnp.exp(x), axis=1)
    t = tgt_ref[0, pl.ds(i * RB, RB)]
    cols = lax.broadcasted_iota(jnp.int32, (RB, C), 1)
    xt = jnp.max(jnp.where(cols == t[:, None], x, -jnp.inf), axis=1)
    loss = jnp.log(s) - xt
    loss_ref[...] = loss.reshape(RB // 128, 128)


def _f32_to_ord_u32(v):
    u = lax.bitcast_convert_type(v, jnp.uint32)
    mask = jnp.where(
        u >= jnp.uint32(0x80000000),
        jnp.uint32(0xFFFFFFFF),
        jnp.uint32(0x80000000),
    )
    return u ^ mask


def _ord_u32_to_f32(t):
    bits = jnp.where(
        t >= jnp.uint32(0x80000000),
        t ^ jnp.uint32(0x80000000),
        ~t,
    )
    return lax.bitcast_convert_type(bits, jnp.float32)


def _combine_body(loss_tc_ref, s16_ref, xt_sc_ref, out_ref):
    loss_tc = loss_tc_ref[...]
    part = s16_ref[...]
    rsel = (
        lax.broadcasted_iota(jnp.int32, (RPW * 16, RPW), 0) // 16
        == lax.broadcasted_iota(jnp.int32, (RPW * 16, RPW), 1)
    ).astype(jnp.float32)
    s2 = jnp.dot(part, rsel, preferred_element_type=jnp.float32)
    loss_sc = jnp.log(s2) - xt_sc_ref[...]
    k_tc = _f32_to_ord_u32(loss_tc)
    k_sc = _f32_to_ord_u32(loss_sc)

    def bit_step(b, acc):
        cand = acc | (jnp.uint32(1) << (jnp.uint32(31) - b.astype(jnp.uint32)))
        cnt = jnp.sum((k_tc >= cand).astype(jnp.int32)) + jnp.sum(
            (k_sc >= cand).astype(jnp.int32)
        )
        return jnp.where(cnt >= K, cand, acc)

    thr = lax.fori_loop(0, 32, bit_step, jnp.uint32(0))
    gt_tc = k_tc > thr
    gt_sc = k_sc > thr
    cnt_gt = jnp.sum(gt_tc.astype(jnp.int32)) + jnp.sum(gt_sc.astype(jnp.int32))
    sum_gt = jnp.sum(jnp.where(gt_tc, loss_tc, jnp.float32(0.0))) + jnp.sum(
        jnp.where(gt_sc, loss_sc, jnp.float32(0.0))
    )
    tval = _ord_u32_to_f32(thr)
    mean = (sum_gt + tval * (K - cnt_gt).astype(jnp.float32)) / jnp.float32(K)
    out_ref[...] = mean.reshape(1, 1)


def kernel(pred, target):
    target = target.astype(jnp.int32)
    s_sc, xt_sc = _sc_rows(pred.reshape(N * C), target)
    loss_tc = pl.pallas_call(
        _tc_body,
        grid=(NBLK,),
        in_specs=[
            pl.BlockSpec((RB, C), lambda i: (i, 0)),
            pl.BlockSpec((1, N), lambda i: (0, 0)),
        ],
        out_specs=pl.BlockSpec((RB // 128, 128), lambda i: (i, 0)),
        out_shape=jax.ShapeDtypeStruct((TLROWS, 128), jnp.float32),
    )(pred, target.reshape(1, N))
    out = pl.pallas_call(
        _combine_body,
        out_shape=jax.ShapeDtypeStruct((1, 1), jnp.float32),
    )(loss_tc, s_sc, xt_sc)
    return out[0, 0]


# speedup vs baseline: 2.7230x; 2.7230x over previous
import functools
import jax
import jax.numpy as jnp
from jax import lax
from jax.experimental import pallas as pl
from jax.experimental.pallas import tpu as pltpu
from jax.experimental.pallas import tpu_sc as plsc

N = 16384
C = 4096
K = int(0.7 * N)

NSC = 4096
NTC = N - NSC
NW = 32
RPW = NSC // NW
CH = 8
NCH = RPW // CH
RB = 1024
NBLK = NTC // RB
TLROWS = NTC // 128
SLROWS = NSC // 128

_mesh = plsc.VectorSubcoreMesh(core_axis_name="c", subcore_axis_name="s")


@functools.partial(
    pl.kernel,
    mesh=_mesh,
    out_type=(
        jax.ShapeDtypeStruct((NW, RPW * 16), jnp.float32),
        jax.ShapeDtypeStruct((NW, RPW * 16), jnp.float32),
    ),
    scratch_types=[
        pltpu.VMEM((CH, C), jnp.float32),
        pltpu.VMEM((CH, C), jnp.float32),
        pltpu.VMEM((RPW,), jnp.int32),
        pltpu.VMEM((RPW * 16,), jnp.float32),
        pltpu.VMEM((RPW * 16,), jnp.float32),
        pltpu.SemaphoreType.DMA,
        pltpu.SemaphoreType.DMA,
    ],
)
def _sc_rows(pred_hbm, tgt_hbm, s16_out, xt16_out, buf0, buf1, tgt_v,
             s16buf, xt16buf, sem0, sem1):
    cid = lax.axis_index("c")
    sid = lax.axis_index("s")
    wid = sid * 2 + cid
    base = NTC + wid * RPW

    pltpu.sync_copy(tgt_hbm.at[pl.ds(base, RPW)], tgt_v)
    lanes = lax.iota(jnp.int32, 16)
    zv = jnp.zeros((16,), jnp.float32)

    def process(buf, ch_idx, tvec, lane_base):
        for r in range(CH):
            def col_body(j, accs):
                a0, a1, a2, a3 = accs
                b = j * 64
                a0 = a0 + jnp.exp(buf[r, pl.ds(b, 16)])
                a1 = a1 + jnp.exp(buf[r, pl.ds(b + 16, 16)])
                a2 = a2 + jnp.exp(buf[r, pl.ds(b + 32, 16)])
                a3 = a3 + jnp.exp(buf[r, pl.ds(b + 48, 16)])
                return (a0, a1, a2, a3)

            a0, a1, a2, a3 = lax.fori_loop(0, C // 64, col_body, (zv, zv, zv, zv))
            off = (ch_idx * CH + r) * 16
            s16buf[pl.ds(off, 16)] = (a0 + a1) + (a2 + a3)
            t = tvec[lane_base + r]
            c0a = pl.multiple_of(jnp.bitwise_and(t, ~15), 16)
            grp = buf[r, pl.ds(c0a, 16)]
            xt16buf[pl.ds(off, 16)] = jnp.where(
                lanes == jnp.bitwise_and(t, 15), grp, zv
            )

    pltpu.async_copy(pred_hbm.at[pl.ds(base, CH)], buf0, sem0)

    def outer(j, carry):
        c0 = 2 * j
        r0 = base + c0 * CH
        tvec = tgt_v[pl.ds(j * 16, 16)]
        pltpu.async_copy(pred_hbm.at[pl.ds(r0 + CH, CH)], buf1, sem1)
        pltpu.make_async_copy(pred_hbm.at[pl.ds(r0, CH)], buf0, sem0).wait()
        process(buf0, c0, tvec, 0)

        @pl.when(c0 + 2 < NCH)
        def _():
            pltpu.async_copy(pred_hbm.at[pl.ds(r0 + 2 * CH, CH)], buf0, sem0)

        pltpu.make_async_copy(pred_hbm.at[pl.ds(r0 + CH, CH)], buf1, sem1).wait()
        process(buf1, c0 + 1, tvec, CH)
        return carry

    lax.fori_loop(0, NCH // 2, outer, 0)
    pltpu.sync_copy(s16buf, s16_out.at[wid])
    pltpu.sync_copy(xt16buf, xt16_out.at[wid])


def _tc_body(pred_ref, tgt_ref, loss_ref):
    i = pl.program_id(0)
    x = pred_ref[...]
    s = jnp.sum(jnp.exp(x), axis=1)
    t = tgt_ref[0, pl.ds(i * RB, RB)]
    cols = lax.broadcasted_iota(jnp.int32, (RB, C), 1)
    xt = jnp.max(jnp.where(cols == t[:, None], x, -jnp.inf), axis=1)
    loss = jnp.log(s) - xt
    loss_ref[...] = loss.reshape(RB // 128, 128)


def _f32_to_ord_u32(v):
    u = lax.bitcast_convert_type(v, jnp.uint32)
    mask = jnp.where(
        u >= jnp.uint32(0x80000000),
        jnp.uint32(0xFFFFFFFF),
        jnp.uint32(0x80000000),
    )
    return u ^ mask


def _ord_u32_to_f32(t):
    bits = jnp.where(
        t >= jnp.uint32(0x80000000),
        t ^ jnp.uint32(0x80000000),
        ~t,
    )
    return lax.bitcast_convert_type(bits, jnp.float32)


def _combine_body(loss_tc_ref, s16_ref, xt16_ref, out_ref):
    loss_tc = loss_tc_ref[...]
    rsel = (
        lax.broadcasted_iota(jnp.int32, (RPW * 16, RPW), 0) // 16
        == lax.broadcasted_iota(jnp.int32, (RPW * 16, RPW), 1)
    ).astype(jnp.float32)
    s2 = jnp.dot(s16_ref[...], rsel, preferred_element_type=jnp.float32)
    xt = jnp.dot(xt16_ref[...], rsel, preferred_element_type=jnp.float32)
    loss_sc = jnp.log(s2) - xt
    k_tc = _f32_to_ord_u32(loss_tc)
    k_sc = _f32_to_ord_u32(loss_sc)

    def bit_step(b, acc):
        cand = acc | (jnp.uint32(1) << (jnp.uint32(31) - b.astype(jnp.uint32)))
        cnt = jnp.sum((k_tc >= cand).astype(jnp.int32)) + jnp.sum(
            (k_sc >= cand).astype(jnp.int32)
        )
        return jnp.where(cnt >= K, cand, acc)

    thr = lax.fori_loop(0, 32, bit_step, jnp.uint32(0))
    gt_tc = k_tc > thr
    gt_sc = k_sc > thr
    cnt_gt = jnp.sum(gt_tc.astype(jnp.int32)) + jnp.sum(gt_sc.astype(jnp.int32))
    sum_gt = jnp.sum(jnp.where(gt_tc, loss_tc, jnp.float32(0.0))) + jnp.sum(
        jnp.where(gt_sc, loss_sc, jnp.float32(0.0))
    )
    tval = _ord_u32_to_f32(thr)
    mean = (sum_gt + tval * (K - cnt_gt).astype(jnp.float32)) / jnp.float32(K)
    out_ref[...] = mean.reshape(1, 1)


def kernel(pred, target):
    target = target.astype(jnp.int32)
    s16_sc, xt16_sc = _sc_rows(pred, target)
    loss_tc = pl.pallas_call(
        _tc_body,
        grid=(NBLK,),
        in_specs=[
            pl.BlockSpec((RB, C), lambda i: (i, 0)),
            pl.BlockSpec((1, N), lambda i: (0, 0)),
        ],
        out_specs=pl.BlockSpec((RB // 128, 128), lambda i: (i, 0)),
        out_shape=jax.ShapeDtypeStruct((TLROWS, 128), jnp.float32),
    )(pred, target.reshape(1, N))
    out = pl.pallas_call(
        _combine_body,
        out_shape=jax.ShapeDtypeStruct((1, 1), jnp.float32),
    )(loss_tc, s16_sc, xt16_sc)
    return out[0, 0]


# speedup vs baseline: 3.2920x; 1.2090x over previous
import jax
import jax.numpy as jnp
from jax import lax
from jax.experimental import pallas as pl
from jax.experimental.pallas import tpu as pltpu

N = 16384
C = 4096
K = int(0.7 * N)
RB = 1024
NBLK = N // RB
LROWS = N // 128


def _f32_to_ord_u32(v):
    u = lax.bitcast_convert_type(v, jnp.uint32)
    mask = jnp.where(
        u >= jnp.uint32(0x80000000),
        jnp.uint32(0xFFFFFFFF),
        jnp.uint32(0x80000000),
    )
    return u ^ mask


def _ord_u32_to_f32(t):
    bits = jnp.where(
        t >= jnp.uint32(0x80000000),
        t ^ jnp.uint32(0x80000000),
        ~t,
    )
    return lax.bitcast_convert_type(bits, jnp.float32)


def _body(pred_l_ref, pred_r_ref, tgt_ref, out_ref, loss_ref):
    i = pl.program_id(0)
    xl = pred_l_ref[...]
    xr = pred_r_ref[...]
    s = jnp.sum(jnp.exp(xl), axis=1) + jnp.sum(jnp.exp(xr), axis=1)
    t = tgt_ref[0, pl.ds(i * RB, RB)]
    cols = lax.broadcasted_iota(jnp.int32, (RB, C // 2), 1)
    tcol = t[:, None]
    neg = jnp.float32(-1e30)
    xt = jnp.maximum(
        jnp.max(jnp.where(cols == tcol, xl, neg), axis=1),
        jnp.max(jnp.where(cols + (C // 2) == tcol, xr, neg), axis=1),
    )
    loss = jnp.log(s) - xt
    r = RB // 128
    loss_ref[pl.ds(i * r, r), :] = loss.reshape(r, 128)

    @pl.when(i == NBLK - 1)
    def _select():
        vals = loss_ref[...]
        keys = _f32_to_ord_u32(vals)

        def bit_step(b, acc):
            cand = acc | (jnp.uint32(1) << (jnp.uint32(31) - b.astype(jnp.uint32)))
            cnt = jnp.sum((keys >= cand).astype(jnp.int32))
            return jnp.where(cnt >= K, cand, acc)

        thr = lax.fori_loop(0, 32, bit_step, jnp.uint32(0))
        gt = keys > thr
        cnt_gt = jnp.sum(gt.astype(jnp.int32))
        sum_gt = jnp.sum(jnp.where(gt, vals, jnp.float32(0.0)))
        tval = _ord_u32_to_f32(thr)
        mean = (sum_gt + tval * (K - cnt_gt).astype(jnp.float32)) / jnp.float32(K)
        out_ref[...] = mean.reshape(1, 1)


def kernel(pred, target):
    target = target.astype(jnp.int32).reshape(1, N)
    out = pl.pallas_call(
        _body,
        grid=(NBLK,),
        in_specs=[
            pl.BlockSpec((RB, C // 2), lambda i: (i, 0)),
            pl.BlockSpec((RB, C // 2), lambda i: (i, 1)),
            pl.BlockSpec((1, N), lambda i: (0, 0)),
        ],
        out_specs=pl.BlockSpec((1, 1), lambda i: (0, 0)),
        out_shape=jax.ShapeDtypeStruct((1, 1), jnp.float32),
        scratch_shapes=[pltpu.VMEM((LROWS, 128), jnp.float32)],
    )(pred, pred, target)
    return out[0, 0]


# speedup vs baseline: 3.3541x; 1.0189x over previous
import jax
import jax.numpy as jnp
from jax import lax
from jax.experimental import pallas as pl
from jax.experimental.pallas import tpu as pltpu

N = 16384
C = 4096
K = int(0.7 * N)
RB = 1024
NBLK = N // RB
LROWS = N // 128


def _f32_to_ord_u32(v):
    u = lax.bitcast_convert_type(v, jnp.uint32)
    mask = jnp.where(
        u >= jnp.uint32(0x80000000),
        jnp.uint32(0xFFFFFFFF),
        jnp.uint32(0x80000000),
    )
    return u ^ mask


def _ord_u32_to_f32(t):
    bits = jnp.where(
        t >= jnp.uint32(0x80000000),
        t ^ jnp.uint32(0x80000000),
        ~t,
    )
    return lax.bitcast_convert_type(bits, jnp.float32)


def _body(p0_ref, p1_ref, p2_ref, p3_ref, tgt_ref, out_ref, loss_ref):
    i = pl.program_id(0)
    q = C // 4
    t = tgt_ref[0, pl.ds(i * RB, RB)]
    tcol = t[:, None]
    cols = lax.broadcasted_iota(jnp.int32, (RB, q), 1)
    neg = jnp.float32(-1e30)
    s = jnp.zeros((RB,), jnp.float32)
    xt = jnp.full((RB,), neg)
    for h, ref in enumerate((p0_ref, p1_ref, p2_ref, p3_ref)):
        x = ref[...]
        s = s + jnp.sum(jnp.exp(x), axis=1)
        xt = jnp.maximum(
            xt, jnp.max(jnp.where(cols + (h * q) == tcol, x, neg), axis=1)
        )
    loss = jnp.log(s) - xt
    r = RB // 128
    loss_ref[pl.ds(i * r, r), :] = loss.reshape(r, 128)

    @pl.when(i == NBLK - 1)
    def _select():
        vals = loss_ref[...]
        keys = _f32_to_ord_u32(vals)

        def bit_step(b, acc):
            cand = acc | (jnp.uint32(1) << (jnp.uint32(31) - b.astype(jnp.uint32)))
            cnt = jnp.sum((keys >= cand).astype(jnp.int32))
            return jnp.where(cnt >= K, cand, acc)

        thr = lax.fori_loop(0, 32, bit_step, jnp.uint32(0))
        gt = keys > thr
        cnt_gt = jnp.sum(gt.astype(jnp.int32))
        sum_gt = jnp.sum(jnp.where(gt, vals, jnp.float32(0.0)))
        tval = _ord_u32_to_f32(thr)
        mean = (sum_gt + tval * (K - cnt_gt).astype(jnp.float32)) / jnp.float32(K)
        out_ref[...] = mean.reshape(1, 1)


def kernel(pred, target):
    target = target.astype(jnp.int32).reshape(1, N)
    out = pl.pallas_call(
        _body,
        grid=(NBLK,),
        in_specs=[
            pl.BlockSpec((RB, C // 4), lambda i: (i, 0)),
            pl.BlockSpec((RB, C // 4), lambda i: (i, 1)),
            pl.BlockSpec((RB, C // 4), lambda i: (i, 2)),
            pl.BlockSpec((RB, C // 4), lambda i: (i, 3)),
            pl.BlockSpec((1, N), lambda i: (0, 0)),
        ],
        out_specs=pl.BlockSpec((1, 1), lambda i: (0, 0)),
        out_shape=jax.ShapeDtypeStruct((1, 1), jnp.float32),
        scratch_shapes=[pltpu.VMEM((LROWS, 128), jnp.float32)],
    )(pred, pred, pred, pred, target)
    return out[0, 0]
